# baseline (speedup 1.0000x reference)
import functools

import jax
import jax.numpy as jnp
from jax import lax
from jax.experimental import pallas as pl
from jax.experimental.pallas import tpu as pltpu
from jax.experimental.pallas import tpu_sc as plsc

NC = 2
NS = 16
LANES = 16
NBUF = 4
GAHEAD = 2


def _pick_chunk(per_tile: int) -> int:
    for c in range(128, 0, -16):
        if per_tile % c == 0:
            return c
    raise ValueError(f"per-tile edge count {per_tile} not divisible by 16")


def _sc_scatter(edge_feat, switch_val, edge_src, edge_dst, n_nodes: int):
    E, D = edge_feat.shape
    NW = NC * NS
    assert E % NW == 0 and D % LANES == 0
    per_tile = E // NW
    chunk = _pick_chunk(per_tile)
    n_chunks = per_tile // chunk
    assert n_chunks > NBUF
    n_pad = -(-n_nodes // (NS * 8)) * (NS * 8)
    rows_out = n_pad // NS
    dreg = D // LANES

    mesh = plsc.VectorSubcoreMesh(core_axis_name="c", subcore_axis_name="s")

    @functools.partial(
        pl.kernel,
        mesh=mesh,
        out_type=pltpu.HBM((NC, n_pad, D), jnp.float32),
        scratch_types=[
            pltpu.VMEM((NBUF, chunk, D), jnp.float32),
            pltpu.VMEM((NBUF, chunk), jnp.float32),
            pltpu.VMEM((NBUF, chunk), jnp.int32),
            pltpu.VMEM((NBUF, chunk), jnp.int32),
            pltpu.SemaphoreType.DMA,
            pltpu.SemaphoreType.DMA,
            pltpu.VMEM_SHARED((n_pad, D), jnp.float32),
        ],
    )
    def k(feat_hbm, sw_hbm, src_hbm, dst_hbm, out_hbm,
          feat2, sw2, src2, dst2, gsem, ssem, acc):
        c = lax.axis_index("c")
        s = lax.axis_index("s")
        wid = c * NS + s
        base = wid * per_tile

        zvec = jnp.zeros((LANES,), jnp.float32)

        def zrow(i, _):
            for d in range(dreg):
                feat2[0, i, pl.ds(d * LANES, LANES)] = zvec
            return 0

        lax.fori_loop(0, chunk, zrow, 0)
        n_full, rem = divmod(rows_out, chunk)
        for z in range(n_full):
            pltpu.sync_copy(feat2.at[0],
                            acc.at[pl.ds(s * rows_out + z * chunk, chunk)])
        if rem:
            pltpu.sync_copy(feat2.at[0, pl.ds(0, rem)],
                            acc.at[pl.ds(s * rows_out + n_full * chunk, rem)])
        plsc.subcore_barrier()

        def gather_copies(kk, b):
            off = pl.ds(base + kk * chunk, chunk)
            return (
                (feat_hbm.at[off], feat2.at[b]),
                (sw_hbm.at[off], sw2.at[b]),
                (src_hbm.at[off], src2.at[b]),
                (dst_hbm.at[off], dst2.at[b]),
            )

        def start_gathers(kk, b):
            for src, dst in gather_copies(kk, b):
                pltpu.async_copy(src, dst, gsem)

        def wait_gathers(kk, b):
            for src, dst in gather_copies(kk, b):
                pltpu.make_async_copy(src, dst, gsem).wait()

        def drain_scatters(b):
            for idx2 in (src2, dst2):
                pltpu.make_async_copy(
                    feat2.at[b], acc.at[idx2.at[b]], ssem).wait()

        def phase(kk, b):
            with jax.named_scope("ph_waitg"):
                wait_gathers(kk, b)

            with jax.named_scope("ph_drain"):
                @pl.when(kk >= NBUF - GAHEAD)
                def _():
                    drain_scatters((b + GAHEAD) % NBUF)

            with jax.named_scope("ph_start"):
                @pl.when(kk < n_chunks - GAHEAD)
                def _():
                    start_gathers(kk + GAHEAD, (b + GAHEAD) % NBUF)

            def mul_group(g, _):
                svec = sw2[b, pl.ds(g * LANES, LANES)]
                for jj in range(LANES):
                    row = g * LANES + jj
                    sval = svec[jj]
                    for d in range(dreg):
                        sl = pl.ds(d * LANES, LANES)
                        feat2[b, row, sl] = feat2[b, row, sl] * sval
                return 0

            with jax.named_scope("ph_mul"):
                lax.fori_loop(0, chunk // LANES, mul_group, 0)
            with jax.named_scope("ph_scat"):
                pltpu.async_copy(feat2.at[b], acc.at[src2.at[b]], ssem, add=True)
                pltpu.async_copy(feat2.at[b], acc.at[dst2.at[b]], ssem, add=True)

        for j in range(GAHEAD):
            start_gathers(j, j % NBUF)
        for p in range(NBUF):
            phase(p, p)

        def grp(i, _):
            for j in range(NBUF):
                phase(NBUF * i + NBUF + j, j)
            return 0

        n_main = (n_chunks - NBUF) // NBUF
        lax.fori_loop(0, n_main, grp, 0)
        for t in range((n_chunks - NBUF) % NBUF):
            kk = NBUF * (n_main + 1) + t
            phase(kk, kk % NBUF)
        for kk in range(n_chunks - (NBUF - GAHEAD), n_chunks):
            drain_scatters(kk % NBUF)

        plsc.subcore_barrier()
        rows = pl.ds(s * rows_out, rows_out)
        pltpu.sync_copy(acc.at[rows], out_hbm.at[c, rows])

    return k(edge_feat, switch_val, edge_src, edge_dst)


def _combine_partials(partials):
    _, n_pad, D = partials.shape
    blk = 1024 if n_pad % 1024 == 0 else n_pad

    def add_k(p_ref, o_ref):
        o_ref[...] = p_ref[0] + p_ref[1]

    return pl.pallas_call(
        add_k,
        grid=(n_pad // blk,),
        in_specs=[pl.BlockSpec((NC, blk, D), lambda i: (0, i, 0))],
        out_specs=pl.BlockSpec((blk, D), lambda i: (i, 0)),
        out_shape=jax.ShapeDtypeStruct((n_pad, D), jnp.float32),
    )(partials)


def kernel(edge_feat, species, edge_src, edge_dst, switch_val):
    n_nodes = species.shape[0]
    partials = _sc_scatter(edge_feat, switch_val, edge_src, edge_dst, n_nodes)
    return _combine_partials(partials)[:n_nodes]


# speedup vs baseline: 1.0299x; 1.0299x over previous
import functools

import jax
import jax.numpy as jnp
from jax import lax
from jax.experimental import pallas as pl
from jax.experimental.pallas import tpu as pltpu
from jax.experimental.pallas import tpu_sc as plsc

NC = 2
NS = 16
LANES = 16


def _pick_chunk(per_tile: int) -> int:
    for c in range(128, 0, -16):
        if per_tile % c == 0:
            return c
    raise ValueError(f"per-tile edge count {per_tile} not divisible by 16")


def _sc_scatter(edge_feat, switch_val, edge_src, edge_dst, n_nodes: int):
    E, D = edge_feat.shape
    NW = NC * NS
    assert E % NW == 0 and D % LANES == 0
    per_tile = E // NW
    chunk = _pick_chunk(per_tile)
    n_chunks = per_tile // chunk
    n_pad = -(-n_nodes // (NS * 8)) * (NS * 8)
    rows_out = n_pad // NS
    dreg = D // LANES

    mesh = plsc.VectorSubcoreMesh(core_axis_name="c", subcore_axis_name="s")

    @functools.partial(
        pl.kernel,
        mesh=mesh,
        out_type=pltpu.HBM((NC, n_pad, D), jnp.float32),
        scratch_types=[
            pltpu.VMEM((2, chunk, D), jnp.float32),
            pltpu.VMEM((2, chunk), jnp.float32),
            pltpu.VMEM((2, chunk), jnp.int32),
            pltpu.VMEM((2, chunk), jnp.int32),
            pltpu.SemaphoreType.DMA,
            pltpu.SemaphoreType.DMA,
            pltpu.VMEM_SHARED((n_pad, D), jnp.float32),
        ],
    )
    def k(feat_hbm, sw_hbm, src_hbm, dst_hbm, out_hbm,
          feat2, sw2, src2, dst2, gsem, ssem, acc):
        c = lax.axis_index("c")
        s = lax.axis_index("s")
        wid = c * NS + s
        base = wid * per_tile

        zvec = jnp.zeros((LANES,), jnp.float32)

        def zrow(i, _):
            for d in range(dreg):
                feat2[0, i, pl.ds(d * LANES, LANES)] = zvec
            return 0

        lax.fori_loop(0, chunk, zrow, 0)
        n_full, rem = divmod(rows_out, chunk)
        for z in range(n_full):
            pltpu.sync_copy(feat2.at[0],
                            acc.at[pl.ds(s * rows_out + z * chunk, chunk)])
        if rem:
            pltpu.sync_copy(feat2.at[0, pl.ds(0, rem)],
                            acc.at[pl.ds(s * rows_out + n_full * chunk, rem)])
        plsc.subcore_barrier()

        def gather_copies(kk, b):
            off = pl.ds(base + kk * chunk, chunk)
            return (
                (feat_hbm.at[off], feat2.at[b]),
                (sw_hbm.at[off], sw2.at[b]),
                (src_hbm.at[off], src2.at[b]),
                (dst_hbm.at[off], dst2.at[b]),
            )

        def start_gathers(kk, b):
            for src, dst in gather_copies(kk, b):
                pltpu.async_copy(src, dst, gsem)

        def wait_gathers(kk, b):
            for src, dst in gather_copies(kk, b):
                pltpu.make_async_copy(src, dst, gsem).wait()

        def drain_scatters(b):
            for idx2 in (src2, dst2):
                pltpu.make_async_copy(
                    feat2.at[b], acc.at[idx2.at[b]], ssem).wait()

        def phase(kk, b):
            wait_gathers(kk, b)

            @pl.when(kk > 0)
            def _():
                drain_scatters(1 - b)

            @pl.when(kk < n_chunks - 1)
            def _():
                start_gathers(kk + 1, 1 - b)

            def mul_group(g, _):
                svec = sw2[b, pl.ds(g * LANES, LANES)]
                for jj in range(LANES):
                    row = g * LANES + jj
                    sval = svec[jj]
                    for d in range(dreg):
                        sl = pl.ds(d * LANES, LANES)
                        feat2[b, row, sl] = feat2[b, row, sl] * sval
                return 0

            lax.fori_loop(0, chunk // LANES, mul_group, 0)
            pltpu.async_copy(feat2.at[b], acc.at[src2.at[b]], ssem, add=True)
            pltpu.async_copy(feat2.at[b], acc.at[dst2.at[b]], ssem, add=True)

        start_gathers(0, 0)
        phase(0, 0)

        def pair(i, _):
            phase(2 * i + 1, 1)
            phase(2 * i + 2, 0)
            return 0

        lax.fori_loop(0, (n_chunks - 1) // 2, pair, 0)
        if (n_chunks - 1) % 2:
            phase(n_chunks - 1, 1)
        drain_scatters((n_chunks - 1) % 2)

        plsc.subcore_barrier()
        rows = pl.ds(s * rows_out, rows_out)
        pltpu.sync_copy(acc.at[rows], out_hbm.at[c, rows])

    return k(edge_feat, switch_val, edge_src, edge_dst)


def _combine_partials(partials):
    _, n_pad, D = partials.shape
    blk = 1024 if n_pad % 1024 == 0 else n_pad

    def add_k(p_ref, o_ref):
        o_ref[...] = p_ref[0] + p_ref[1]

    return pl.pallas_call(
        add_k,
        grid=(n_pad // blk,),
        in_specs=[pl.BlockSpec((NC, blk, D), lambda i: (0, i, 0))],
        out_specs=pl.BlockSpec((blk, D), lambda i: (i, 0)),
        out_shape=jax.ShapeDtypeStruct((n_pad, D), jnp.float32),
    )(partials)


def kernel(edge_feat, species, edge_src, edge_dst, switch_val):
    n_nodes = species.shape[0]
    partials = _sc_scatter(edge_feat, switch_val, edge_src, edge_dst, n_nodes)
    return _combine_partials(partials)[:n_nodes]
